# baseline (speedup 1.0000x reference)
import functools

import numpy as np

import jax
import jax.numpy as jnp
from jax import lax
from jax.experimental import pallas as pl
from jax.experimental.pallas import tpu as pltpu
from jax.experimental.pallas import tpu_sc as plsc

N_WORKERS = 32
NODES_PER_WORKER = 320
BLOCK = 256
SLAB = 328
TRASH = SLAB - 1


_SHIFT = BLOCK.bit_length() - 1
assert (1 << _SHIFT) == BLOCK


def _sc_segment_sum(i, idx, cuts, n_nodes, n_feat):
    mesh = plsc.VectorSubcoreMesh(core_axis_name="c", subcore_axis_name="s")

    @functools.partial(
        pl.kernel,
        out_type=jax.ShapeDtypeStruct((n_nodes, n_feat), jnp.float32),
        mesh=mesh,
        scratch_types=[
            pltpu.VMEM((BLOCK, n_feat), jnp.float32),
            pltpu.VMEM((BLOCK, n_feat), jnp.float32),
            pltpu.VMEM((BLOCK,), jnp.int32),
            pltpu.VMEM((BLOCK,), jnp.int32),
            pltpu.VMEM((2, 128), jnp.int32),
            pltpu.VMEM((2, 128), jnp.int32),
            pltpu.VMEM((16,), jnp.int32),
            pltpu.VMEM_SHARED((16 * SLAB, n_feat), jnp.float32),
            pltpu.SemaphoreType.DMA,
            pltpu.SemaphoreType.DMA,
            pltpu.SemaphoreType.DMA,
            pltpu.SemaphoreType.DMA,
        ],
    )
    def body(i_hbm, idx_hbm, cuts_hbm, out_hbm,
             ebuf0, ebuf1, ibuf0, ibuf1, lidx0, lidx1, cbuf, acc,
             lsem0, lsem1, ssem0, ssem1):
        c = lax.axis_index("c")
        s = lax.axis_index("s")
        wid = c * 16 + s
        slab = pl.multiple_of(s * SLAB, 8)

        pltpu.sync_copy(cuts_hbm.at[pl.ds(pl.multiple_of(wid * 16, 8), 16)], cbuf)
        pv = cbuf[...]
        n0 = pl.multiple_of(wid * NODES_PER_WORKER, 8)
        blk0 = lax.shift_right_logical(pv[0], _SHIFT)
        blkend = lax.shift_right_logical(pv[1] + (BLOCK - 1), _SHIFT)
        nblk = blkend - blk0
        ebufs = (ebuf0, ebuf1)
        ibufs = (ibuf0, ibuf1)
        lidxs = (lidx0, lidx1)
        lsems = (lsem0, lsem1)
        ssems = (ssem0, ssem1)

        def copies(b, k):
            e0 = pl.multiple_of(b * BLOCK, 8)
            return (
                pltpu.make_async_copy(i_hbm.at[pl.ds(e0, BLOCK)], ebufs[k], lsems[k]),
                pltpu.make_async_copy(idx_hbm.at[pl.ds(e0, BLOCK)], ibufs[k], lsems[k]),
            )

        @pl.when(nblk > 0)
        def _():
            for cp in copies(blk0, 0):
                cp.start()

        def zrow(r, carry):
            for cc in range(n_feat // 16):
                ebuf1[r, pl.ds(cc * 16, 16)] = jnp.zeros((16,), jnp.float32)
            return carry

        lax.fori_loop(0, BLOCK, zrow, 0)
        pltpu.sync_copy(ebuf1, acc.at[pl.ds(slab, BLOCK)])
        pltpu.sync_copy(
            ebuf1.at[pl.ds(0, SLAB - BLOCK)],
            acc.at[pl.ds(slab + BLOCK, SLAB - BLOCK)],
        )

        base = n0 - slab

        def drain_scatter(k):
            for j in range(BLOCK // 128):
                pltpu.make_async_copy(
                    i_hbm.at[pl.ds(0, 128)],
                    ebufs[k].at[pl.ds(j * 128, 128)],
                    ssems[k],
                ).wait()

        def process(b, k):
            for cp in copies(b, k):
                cp.wait()

            ibuf = ibufs[k]
            lidx = lidxs[k]
            for kk in range(BLOCK // 16):
                iv = ibuf[pl.ds(kk * 16, 16)]
                owned = (iv >= n0) & (iv < n0 + NODES_PER_WORKER)
                lv = jnp.where(owned, iv - base, slab + TRASH)
                lidx[kk // 8, pl.ds((kk % 8) * 16, 16)] = lv

            @pl.when(b + 1 < blkend)
            def _():
                @pl.when(b > blk0)
                def _():
                    drain_scatter(1 - k)

                for cp in copies(b + 1, 1 - k):
                    cp.start()

            for j in range(BLOCK // 128):
                pltpu.async_copy(
                    ebufs[k].at[pl.ds(j * 128, 128)],
                    acc.at[lidx.at[j]],
                    ssems[k],
                    add=True,
                )

        def pair(t, carry):
            b = blk0 + 2 * t
            process(b, 0)

            @pl.when(b + 1 < blkend)
            def _():
                process(b + 1, 1)

            return carry

        lax.fori_loop(0, (nblk + 1) // 2, pair, 0)

        @pl.when(nblk == 1)
        def _():
            drain_scatter(0)

        @pl.when(nblk >= 2)
        def _():
            drain_scatter(0)
            drain_scatter(1)

        last_rows = n_nodes - (N_WORKERS - 1) * NODES_PER_WORKER

        @pl.when(wid != N_WORKERS - 1)
        def _():
            pltpu.sync_copy(
                acc.at[pl.ds(slab, NODES_PER_WORKER)],
                out_hbm.at[pl.ds(n0, NODES_PER_WORKER)],
            )

        @pl.when(wid == N_WORKERS - 1)
        def _():
            pltpu.sync_copy(
                acc.at[pl.ds(slab, last_rows)],
                out_hbm.at[pl.ds(n0, last_rows)],
            )

    return body(i, idx, cuts)


def kernel(i, idx_i, p):
    n_nodes, n_feat = p.shape
    idx = idx_i.astype(jnp.int32)
    bounds = jnp.arange(N_WORKERS + 1, dtype=jnp.int32) * NODES_PER_WORKER
    cuts = jnp.sum((idx[:, None] < bounds[None, :]).astype(jnp.int32), axis=0)
    pattern = np.zeros((N_WORKERS, 16), np.int32)
    pattern[:, 0] = np.arange(N_WORKERS)
    pattern[:, 1] = np.arange(N_WORKERS) + 1
    params = cuts[jnp.asarray(pattern.reshape(-1))]
    return _sc_segment_sum(i, idx, params, n_nodes, n_feat)


# speedup vs baseline: 1.1720x; 1.1720x over previous
import functools

import numpy as np

import jax
import jax.numpy as jnp
from jax import lax
from jax.experimental import pallas as pl
from jax.experimental.pallas import tpu as pltpu
from jax.experimental.pallas import tpu_sc as plsc

N_WORKERS = 32
NODES_PER_WORKER = 320
BLOCK = 256
SLAB = 328
TRASH = SLAB - 1


_SHIFT = BLOCK.bit_length() - 1
assert (1 << _SHIFT) == BLOCK


def _sc_segment_sum(i, idx, cuts, n_nodes, n_feat):
    mesh = plsc.VectorSubcoreMesh(core_axis_name="c", subcore_axis_name="s")

    @functools.partial(
        pl.kernel,
        out_type=jax.ShapeDtypeStruct((n_nodes, n_feat), jnp.float32),
        mesh=mesh,
        scratch_types=[
            pltpu.VMEM((BLOCK, n_feat), jnp.float32),
            pltpu.VMEM((BLOCK, n_feat), jnp.float32),
            pltpu.VMEM((BLOCK,), jnp.int32),
            pltpu.VMEM((BLOCK,), jnp.int32),
            pltpu.VMEM((2, 128), jnp.int32),
            pltpu.VMEM((2, 128), jnp.int32),
            pltpu.VMEM((16,), jnp.int32),
            pltpu.VMEM_SHARED((16 * SLAB, n_feat), jnp.float32),
            pltpu.SemaphoreType.DMA,
            pltpu.SemaphoreType.DMA,
            pltpu.SemaphoreType.DMA,
            pltpu.SemaphoreType.DMA,
        ],
    )
    def body(i_hbm, idx_hbm, cuts_hbm, out_hbm,
             ebuf0, ebuf1, ibuf0, ibuf1, lidx0, lidx1, cbuf, acc,
             lsem0, lsem1, ssem0, ssem1):
        c = lax.axis_index("c")
        s = lax.axis_index("s")
        wid = c * 16 + s
        slab = pl.multiple_of(s * SLAB, 8)

        pltpu.sync_copy(cuts_hbm.at[pl.ds(pl.multiple_of(wid * 16, 8), 16)], cbuf)
        pv = cbuf[...]
        n0 = pl.multiple_of(wid * NODES_PER_WORKER, 8)
        blk0 = lax.shift_right_logical(pv[0], _SHIFT)
        blkend = lax.shift_right_logical(pv[1] + (BLOCK - 1), _SHIFT)
        nblk = blkend - blk0
        ebufs = (ebuf0, ebuf1)
        ibufs = (ibuf0, ibuf1)
        lidxs = (lidx0, lidx1)
        lsems = (lsem0, lsem1)
        ssems = (ssem0, ssem1)

        def copies(b, k):
            e0 = pl.multiple_of(b * BLOCK, 8)
            return (
                pltpu.make_async_copy(i_hbm.at[pl.ds(e0, BLOCK)], ebufs[k], lsems[k]),
                pltpu.make_async_copy(idx_hbm.at[pl.ds(e0, BLOCK)], ibufs[k], lsems[k]),
            )

        @pl.when(nblk > 0)
        def _():
            for cp in copies(blk0, 0):
                cp.start()

        def zrow(r, carry):
            for cc in range(n_feat // 16):
                ebuf1[r, pl.ds(cc * 16, 16)] = jnp.zeros((16,), jnp.float32)
            return carry

        lax.fori_loop(0, BLOCK, zrow, 0)
        pltpu.sync_copy(ebuf1, acc.at[pl.ds(slab, BLOCK)])
        pltpu.sync_copy(
            ebuf1.at[pl.ds(0, SLAB - BLOCK)],
            acc.at[pl.ds(slab + BLOCK, SLAB - BLOCK)],
        )

        base = n0 - slab

        def drain_scatter(k):
            for j in range(BLOCK // 128):
                pltpu.make_async_copy(
                    i_hbm.at[pl.ds(0, 128)],
                    ebufs[k].at[pl.ds(j * 128, 128)],
                    ssems[k],
                ).wait()

        def process(b, k):
            for cp in copies(b, k):
                cp.wait()

            ibuf = ibufs[k]
            lidx = lidxs[k]
            for kk in range(BLOCK // 16):
                iv = ibuf[pl.ds(kk * 16, 16)]
                owned = (iv >= n0) & (iv < n0 + NODES_PER_WORKER)
                lv = jnp.where(owned, iv - base, slab + TRASH)
                lidx[kk // 8, pl.ds((kk % 8) * 16, 16)] = lv

            @pl.when(b + 1 < blkend)
            def _():
                @pl.when(b > blk0)
                def _():
                    drain_scatter(1 - k)

                for cp in copies(b + 1, 1 - k):
                    cp.start()

            for j in range(BLOCK // 128):
                pltpu.async_copy(
                    ebufs[k].at[pl.ds(j * 128, 128)],
                    acc.at[lidx.at[j]],
                    ssems[k],
                    add=True,
                )

        def pair(t, carry):
            b = blk0 + 2 * t
            process(b, 0)

            @pl.when(b + 1 < blkend)
            def _():
                process(b + 1, 1)

            return carry

        lax.fori_loop(0, (nblk + 1) // 2, pair, 0)

        @pl.when(nblk == 1)
        def _():
            drain_scatter(0)

        @pl.when(nblk >= 2)
        def _():
            drain_scatter(0)
            drain_scatter(1)

        last_rows = n_nodes - (N_WORKERS - 1) * NODES_PER_WORKER

        @pl.when(wid != N_WORKERS - 1)
        def _():
            pltpu.sync_copy(
                acc.at[pl.ds(slab, NODES_PER_WORKER)],
                out_hbm.at[pl.ds(n0, NODES_PER_WORKER)],
            )

        @pl.when(wid == N_WORKERS - 1)
        def _():
            pltpu.sync_copy(
                acc.at[pl.ds(slab, last_rows)],
                out_hbm.at[pl.ds(n0, last_rows)],
            )

    return body(i, idx, cuts)


def kernel(i, idx_i, p):
    n_nodes, n_feat = p.shape
    idx = idx_i.astype(jnp.int32)
    bounds = jnp.arange(N_WORKERS + 1, dtype=jnp.int32) * NODES_PER_WORKER
    cuts = jnp.sum((idx[:, None] < bounds[None, :]).astype(jnp.int32), axis=0)
    params = jnp.concatenate(
        [cuts[:N_WORKERS, None], cuts[1:, None],
         jnp.zeros((N_WORKERS, 14), jnp.int32)],
        axis=1,
    ).reshape(N_WORKERS * 16)
    return _sc_segment_sum(i, idx, params, n_nodes, n_feat)


# speedup vs baseline: 1.3233x; 1.1291x over previous
import functools

import jax
import jax.numpy as jnp
from jax import lax
from jax.experimental import pallas as pl
from jax.experimental.pallas import tpu as pltpu
from jax.experimental.pallas import tpu_sc as plsc

N_WORKERS = 32
NODES_PER_WORKER = 320
BLOCK = 128
SLAB = 328
TRASH = SLAB - 1
NBUF = 3

_SHIFT = BLOCK.bit_length() - 1
assert (1 << _SHIFT) == BLOCK


def _sc_segment_sum(i, idx, params, n_nodes, n_feat):
    mesh = plsc.VectorSubcoreMesh(core_axis_name="c", subcore_axis_name="s")

    @functools.partial(
        pl.kernel,
        out_type=jax.ShapeDtypeStruct((n_nodes, n_feat), jnp.float32),
        mesh=mesh,
        scratch_types=(
            [pltpu.VMEM((BLOCK, n_feat), jnp.float32) for _ in range(NBUF)]
            + [pltpu.VMEM((BLOCK,), jnp.int32) for _ in range(NBUF)]
            + [pltpu.VMEM((max(BLOCK // 128, 1), 128), jnp.int32) for _ in range(NBUF)]
            + [
                pltpu.VMEM((16,), jnp.int32),
                pltpu.VMEM_SHARED((16 * SLAB, n_feat), jnp.float32),
            ]
            + [pltpu.SemaphoreType.DMA for _ in range(2 * NBUF)]
        ),
    )
    def body(i_hbm, idx_hbm, prm_hbm, out_hbm, *scratch):
        ebufs = scratch[:NBUF]
        ibufs = scratch[NBUF:2 * NBUF]
        lidxs = scratch[2 * NBUF:3 * NBUF]
        cbuf = scratch[3 * NBUF]
        acc = scratch[3 * NBUF + 1]
        lsems = scratch[3 * NBUF + 2:3 * NBUF + 2 + NBUF]
        ssems = scratch[3 * NBUF + 2 + NBUF:]

        c = lax.axis_index("c")
        s = lax.axis_index("s")
        wid = c * 16 + s
        slab = pl.multiple_of(s * SLAB, 8)

        pltpu.sync_copy(prm_hbm.at[pl.ds(pl.multiple_of(wid * 16, 8), 16)], cbuf)
        pv = cbuf[...]
        n0 = pl.multiple_of(wid * NODES_PER_WORKER, 8)
        blk0 = lax.shift_right_logical(pv[0], _SHIFT)
        blkend = lax.shift_right_logical(pv[1] + (BLOCK - 1), _SHIFT)
        nblk = blkend - blk0

        def copies(b, k):
            e0 = pl.multiple_of(b * BLOCK, 8)
            return (
                pltpu.make_async_copy(i_hbm.at[pl.ds(e0, BLOCK)], ebufs[k], lsems[k]),
                pltpu.make_async_copy(idx_hbm.at[pl.ds(e0, BLOCK)], ibufs[k], lsems[k]),
            )

        for kk in range(NBUF - 1):
            @pl.when(nblk > kk)
            def _():
                for cp in copies(blk0 + kk, kk):
                    cp.start()

        zbuf = ebufs[NBUF - 1]

        def zrow(r, carry):
            for cc in range(n_feat // 16):
                zbuf[r, pl.ds(cc * 16, 16)] = jnp.zeros((16,), jnp.float32)
            return carry

        lax.fori_loop(0, BLOCK, zrow, 0)
        r0 = 0
        while r0 < SLAB:
            chunk = min(BLOCK, SLAB - r0)
            pltpu.sync_copy(
                zbuf.at[pl.ds(0, chunk)],
                acc.at[pl.ds(slab + r0, chunk)],
            )
            r0 += chunk

        base = n0 - slab

        def drain_scatter(k):
            for j in range(BLOCK // 128):
                pltpu.make_async_copy(
                    i_hbm.at[pl.ds(0, 128)],
                    ebufs[k].at[pl.ds(j * 128, 128)],
                    ssems[k],
                ).wait()

        def process(b, k):
            for cp in copies(b, k):
                cp.wait()

            ibuf = ibufs[k]
            lidx = lidxs[k]
            for kk in range(BLOCK // 16):
                iv = ibuf[pl.ds(kk * 16, 16)]
                owned = (iv >= n0) & (iv < n0 + NODES_PER_WORKER)
                lv = jnp.where(owned, iv - base, slab + TRASH)
                lidx[kk // 8, pl.ds((kk % 8) * 16, 16)] = lv

            @pl.when(b + (NBUF - 1) < blkend)
            def _():
                @pl.when(b > blk0)
                def _():
                    drain_scatter((k + NBUF - 1) % NBUF)

                for cp in copies(b + (NBUF - 1), (k + NBUF - 1) % NBUF):
                    cp.start()

            for j in range(BLOCK // 128):
                pltpu.async_copy(
                    ebufs[k].at[pl.ds(j * 128, 128)],
                    acc.at[lidx.at[j]],
                    ssems[k],
                    add=True,
                )

        def ring(t, carry):
            b = blk0 + NBUF * t
            process(b, 0)
            for kk in range(1, NBUF):
                @pl.when(b + kk < blkend)
                def _():
                    process(b + kk, kk)
            return carry

        lax.fori_loop(0, (nblk + NBUF - 1) // NBUF, ring, 0)

        for kk in range(NBUF):
            @pl.when(nblk > kk)
            def _():
                drain_scatter(kk)

        last_rows = n_nodes - (N_WORKERS - 1) * NODES_PER_WORKER

        @pl.when(wid != N_WORKERS - 1)
        def _():
            pltpu.sync_copy(
                acc.at[pl.ds(slab, NODES_PER_WORKER)],
                out_hbm.at[pl.ds(n0, NODES_PER_WORKER)],
            )

        @pl.when(wid == N_WORKERS - 1)
        def _():
            pltpu.sync_copy(
                acc.at[pl.ds(slab, last_rows)],
                out_hbm.at[pl.ds(n0, last_rows)],
            )

    return body(i, idx, params)


def kernel(i, idx_i, p):
    n_nodes, n_feat = p.shape
    idx = idx_i.astype(jnp.int32)
    bounds = jnp.arange(N_WORKERS + 1, dtype=jnp.int32) * NODES_PER_WORKER
    cuts = jnp.sum((idx[:, None] < bounds[None, :]).astype(jnp.int32), axis=0)
    params = jnp.concatenate(
        [cuts[:N_WORKERS, None], cuts[1:, None],
         jnp.zeros((N_WORKERS, 14), jnp.int32)],
        axis=1,
    ).reshape(N_WORKERS * 16)
    return _sc_segment_sum(i, idx, params, n_nodes, n_feat)
